# baseline (speedup 1.0000x reference)
import functools

import jax
import jax.numpy as jnp
from jax import lax
from jax.experimental import pallas as pl
from jax.experimental.pallas import tpu as pltpu
from jax.experimental.pallas import tpu_sc as plsc

N = 10000
K = 16
CIN = 128
COUT = 32
M = 4

NW = 32
N_PAD = 10240
C = 8
PER_W0 = 640
PER_W1 = 0
PER_W_MAX = PER_W0
CORE1_BASE = PER_W0 * 16


def _mm_body(x_ref, w_ref, y_ref):
    y_ref[...] = jnp.dot(x_ref[...], w_ref[...],
                         preferred_element_type=jnp.float32)


def _tc_matmul(x2, wcat):
    blk = 2048
    return pl.pallas_call(
        _mm_body,
        grid=(N_PAD // blk,),
        in_specs=[pl.BlockSpec((blk, CIN), lambda i: (i, 0)),
                  pl.BlockSpec((CIN, 256), lambda i: (0, 0)),],
        out_specs=pl.BlockSpec((blk, 256), lambda i: (i, 0)),
        out_shape=jax.ShapeDtypeStruct((N_PAD, 256), jnp.float32),
    )(x2, wcat)


def _sc_body(wx_hbm, uxf_hbm, adjf_hbm, c_hbm, b_hbm, out_hbm,
             idx_a, idx_b, adj_all, wrows_a, wrows_b, uxf_v, out_all,
             cvec, bvec, sem_a, sem_b):
    cid = lax.axis_index("c")
    sid = lax.axis_index("s")
    pltpu.sync_copy(c_hbm, cvec)
    pltpu.sync_copy(b_hbm, bvec)
    pltpu.sync_copy(uxf_hbm, uxf_v)
    cv = cvec[...]
    cs = [cv[m] for m in range(M)]
    b_lo = bvec[pl.ds(0, 16)]
    b_hi = bvec[pl.ds(16, 16)]

    def worker(base_w, per_w):
        n_pairs = per_w // C // 2
        pltpu.sync_copy(adjf_hbm.at[pl.ds(base_w * K, per_w * K)],
                        adj_all.at[pl.ds(0, per_w * K)])

        def build_idx(idx_ref, ci):
            for cc in range(C):
                a = adj_all[pl.ds((ci * C + cc) * K, K)]
                idx_ref[pl.ds(cc * K, K)] = jnp.maximum(a - 1, 0)

        def compute_chunk(wrows, ci):
            def node_body(cc, _):
                loc = ci * C + cc
                a = adj_all[pl.ds(loc * K, K)]
                valid = a > 0
                cnt = jnp.zeros((16,), jnp.float32) + jnp.sum(
                    jnp.where(valid, 1.0, 0.0))
                invc = jnp.where(cnt > 0.0, 1.0 / cnt, 0.0)
                idx0 = jnp.maximum(a - 1, 0)
                base4 = idx0 * M
                own = (base_w + loc) * M
                ps = []
                for m in range(M):
                    uxg = plsc.load_gather(uxf_v, [base4 + m])
                    uo = plsc.load_gather(
                        uxf_v, [jnp.full((16,), m, jnp.int32) + own])
                    ps.append((uo + cs[m]) - uxg)
                pmax = jnp.maximum(jnp.maximum(ps[0], ps[1]),
                                   jnp.maximum(ps[2], ps[3]))
                es = [jnp.exp(p - pmax) for p in ps]
                ssum = (es[0] + es[1]) + (es[2] + es[3])
                scale = invc / ssum
                wms = [jnp.where(valid, e * scale, 0.0) for e in es]
                acc_lo = b_lo
                acc_hi = b_hi
                for k in range(K):
                    j = cc * K + k
                    for m in range(M):
                        w = wms[m][k]
                        acc_lo = acc_lo + w * wrows[j, pl.ds(32 * m, 16)]
                        acc_hi = acc_hi + w * wrows[j, pl.ds(32 * m + 16, 16)]
                out_all[pl.ds(loc * COUT, 16)] = acc_lo
                out_all[pl.ds(loc * COUT + 16, 16)] = acc_hi
                return 0

            lax.fori_loop(0, C, node_body, 0)

        def pair_body(i, _):
            build_idx(idx_b, 2 * i + 1)
            cp_b = pltpu.async_copy(wx_hbm.at[idx_b], wrows_b, sem_b)
            pltpu.make_async_copy(wx_hbm.at[idx_a], wrows_a, sem_a).wait()
            compute_chunk(wrows_a, 2 * i)

            @pl.when(i < n_pairs - 1)
            def _():
                build_idx(idx_a, 2 * i + 2)
                pltpu.async_copy(wx_hbm.at[idx_a], wrows_a, sem_a)

            cp_b.wait()
            compute_chunk(wrows_b, 2 * i + 1)
            return 0

        build_idx(idx_a, 0)
        pltpu.async_copy(wx_hbm.at[idx_a], wrows_a, sem_a)
        lax.fori_loop(0, n_pairs, pair_body, 0)
        pltpu.sync_copy(out_all.at[pl.ds(0, per_w * COUT)],
                        out_hbm.at[pl.ds(base_w * COUT, per_w * COUT)])

    @pl.when(cid == 0)
    def _():
        worker(sid * PER_W0, PER_W0)


_sc_kernel = functools.partial(
    pl.kernel,
    mesh=plsc.VectorSubcoreMesh(core_axis_name="c", subcore_axis_name="s"),
    compiler_params=pltpu.CompilerParams(needs_layout_passes=False),
    out_type=jax.ShapeDtypeStruct((N_PAD * COUT,), jnp.float32),
    scratch_types=[
        pltpu.VMEM((C * K,), jnp.int32),
        pltpu.VMEM((C * K,), jnp.int32),
        pltpu.VMEM((PER_W_MAX * K,), jnp.int32),
        pltpu.VMEM((C * K, CIN), jnp.float32),
        pltpu.VMEM((C * K, CIN), jnp.float32),
        pltpu.VMEM((N_PAD * M,), jnp.float32),
        pltpu.VMEM((PER_W_MAX * COUT,), jnp.float32),
        pltpu.VMEM((16,), jnp.float32),
        pltpu.VMEM((COUT,), jnp.float32),
        pltpu.SemaphoreType.DMA,
        pltpu.SemaphoreType.DMA,
    ],
)(_sc_body)


def kernel(x, adj, W, b, u, c):
    x2 = x[0]
    x2p = jnp.pad(x2, ((0, N_PAD - N), (0, 0)))
    Wr = W.reshape(M * COUT, CIN)
    wcat = jnp.concatenate(
        [Wr.T, u.T, jnp.zeros((CIN, 256 - M * COUT - M), jnp.float32)],
        axis=1)
    y = _tc_matmul(x2p, wcat)
    wx = y[:, :M * COUT]
    uxf = y[:, M * COUT:M * COUT + M].reshape(-1)
    adjf = jnp.pad(adj, ((0, N_PAD - N), (0, 0))).reshape(-1)
    c_pad = jnp.pad(c, (0, 16 - M))
    out = _sc_kernel(wx, uxf, adjf, c_pad, b)
    return out[:N * COUT].reshape(1, N, COUT)


# speedup vs baseline: 1.0043x; 1.0043x over previous
import functools

import jax
import jax.numpy as jnp
from jax import lax
from jax.experimental import pallas as pl
from jax.experimental.pallas import tpu as pltpu
from jax.experimental.pallas import tpu_sc as plsc

N = 10000
K = 16
CIN = 128
COUT = 32
M = 4

NW = 32
N_PAD = 10240
C = 8
PER_W0 = 640
PER_W1 = 0
PER_W_MAX = PER_W0
CORE1_BASE = PER_W0 * 16


def _mm_body(x_ref, w_ref, y_ref):
    y_ref[...] = jnp.dot(x_ref[...], w_ref[...],
                         preferred_element_type=jnp.float32)


def _tc_matmul(x2, wcat):
    blk = 2048
    return pl.pallas_call(
        _mm_body,
        grid=(N_PAD // blk,),
        in_specs=[pl.BlockSpec((blk, CIN), lambda i: (i, 0)),
                  pl.BlockSpec((CIN, 256), lambda i: (0, 0)),],
        out_specs=pl.BlockSpec((blk, 256), lambda i: (i, 0)),
        out_shape=jax.ShapeDtypeStruct((N_PAD, 256), jnp.float32),
    )(x2, wcat)


def _sc_body(wx_hbm, uxf_hbm, adjf_hbm, c_hbm, b_hbm, out_hbm,
             idx_a, idx_b, adj_all, wrows_a, wrows_b, uxf_v, ux_sh, out_all,
             cvec, bvec, sem_a, sem_b):
    cid = lax.axis_index("c")
    sid = lax.axis_index("s")
    pltpu.sync_copy(c_hbm, cvec)
    pltpu.sync_copy(b_hbm, bvec)

    @pl.when(sid == 0)
    def _():
        pltpu.sync_copy(uxf_hbm, ux_sh)

    plsc.subcore_barrier()
    pltpu.sync_copy(ux_sh, uxf_v)
    cv = cvec[...]
    cs = [cv[m] for m in range(M)]
    b_lo = bvec[pl.ds(0, 16)]
    b_hi = bvec[pl.ds(16, 16)]

    def worker(base_w, per_w):
        n_pairs = per_w // C // 2
        pltpu.sync_copy(adjf_hbm.at[pl.ds(base_w * K, per_w * K)],
                        adj_all.at[pl.ds(0, per_w * K)])

        def build_idx(idx_ref, ci):
            for cc in range(C):
                a = adj_all[pl.ds((ci * C + cc) * K, K)]
                idx_ref[pl.ds(cc * K, K)] = jnp.maximum(a - 1, 0)

        def compute_chunk(wrows, ci):
            def node_body(cc, _):
                loc = ci * C + cc
                a = adj_all[pl.ds(loc * K, K)]
                valid = a > 0
                cnt = jnp.zeros((16,), jnp.float32) + jnp.sum(
                    jnp.where(valid, 1.0, 0.0))
                invc = jnp.where(cnt > 0.0, 1.0 / cnt, 0.0)
                idx0 = jnp.maximum(a - 1, 0)
                base4 = idx0 * M
                own = (base_w + loc) * M
                ps = []
                for m in range(M):
                    uxg = plsc.load_gather(uxf_v, [base4 + m])
                    uo = plsc.load_gather(
                        uxf_v, [jnp.full((16,), m, jnp.int32) + own])
                    ps.append((uo + cs[m]) - uxg)
                pmax = jnp.maximum(jnp.maximum(ps[0], ps[1]),
                                   jnp.maximum(ps[2], ps[3]))
                es = [jnp.exp(p - pmax) for p in ps]
                ssum = (es[0] + es[1]) + (es[2] + es[3])
                scale = invc / ssum
                wms = [jnp.where(valid, e * scale, 0.0) for e in es]
                acc_lo = b_lo
                acc_hi = b_hi
                for k in range(K):
                    j = cc * K + k
                    for m in range(M):
                        w = wms[m][k]
                        acc_lo = acc_lo + w * wrows[j, pl.ds(32 * m, 16)]
                        acc_hi = acc_hi + w * wrows[j, pl.ds(32 * m + 16, 16)]
                out_all[pl.ds(loc * COUT, 16)] = acc_lo
                out_all[pl.ds(loc * COUT + 16, 16)] = acc_hi
                return 0

            lax.fori_loop(0, C, node_body, 0)

        def pair_body(i, _):
            build_idx(idx_b, 2 * i + 1)
            cp_b = pltpu.async_copy(wx_hbm.at[idx_b], wrows_b, sem_b)
            pltpu.make_async_copy(wx_hbm.at[idx_a], wrows_a, sem_a).wait()
            compute_chunk(wrows_a, 2 * i)

            @pl.when(i < n_pairs - 1)
            def _():
                build_idx(idx_a, 2 * i + 2)
                pltpu.async_copy(wx_hbm.at[idx_a], wrows_a, sem_a)

            cp_b.wait()
            compute_chunk(wrows_b, 2 * i + 1)
            return 0

        build_idx(idx_a, 0)
        pltpu.async_copy(wx_hbm.at[idx_a], wrows_a, sem_a)
        lax.fori_loop(0, n_pairs, pair_body, 0)
        pltpu.sync_copy(out_all.at[pl.ds(0, per_w * COUT)],
                        out_hbm.at[pl.ds(base_w * COUT, per_w * COUT)])

    @pl.when(cid == 0)
    def _():
        worker(sid * PER_W0, PER_W0)


_sc_kernel = functools.partial(
    pl.kernel,
    mesh=plsc.VectorSubcoreMesh(core_axis_name="c", subcore_axis_name="s"),
    compiler_params=pltpu.CompilerParams(needs_layout_passes=False),
    out_type=jax.ShapeDtypeStruct((N_PAD * COUT,), jnp.float32),
    scratch_types=[
        pltpu.VMEM((C * K,), jnp.int32),
        pltpu.VMEM((C * K,), jnp.int32),
        pltpu.VMEM((PER_W_MAX * K,), jnp.int32),
        pltpu.VMEM((C * K, CIN), jnp.float32),
        pltpu.VMEM((C * K, CIN), jnp.float32),
        pltpu.VMEM((N_PAD * M,), jnp.float32),
        pltpu.VMEM_SHARED((N_PAD * M,), jnp.float32),
        pltpu.VMEM((PER_W_MAX * COUT,), jnp.float32),
        pltpu.VMEM((16,), jnp.float32),
        pltpu.VMEM((COUT,), jnp.float32),
        pltpu.SemaphoreType.DMA,
        pltpu.SemaphoreType.DMA,
    ],
)(_sc_body)


def kernel(x, adj, W, b, u, c):
    x2 = x[0]
    x2p = jnp.pad(x2, ((0, N_PAD - N), (0, 0)))
    Wr = W.reshape(M * COUT, CIN)
    wcat = jnp.concatenate(
        [Wr.T, u.T, jnp.zeros((CIN, 256 - M * COUT - M), jnp.float32)],
        axis=1)
    y = _tc_matmul(x2p, wcat)
    wx = y[:, :M * COUT]
    uxf = y[:, M * COUT:M * COUT + M].reshape(-1)
    adjf = jnp.pad(adj, ((0, N_PAD - N), (0, 0))).reshape(-1)
    c_pad = jnp.pad(c, (0, 16 - M))
    out = _sc_kernel(wx, uxf, adjf, c_pad, b)
    return out[:N * COUT].reshape(1, N, COUT)


# speedup vs baseline: 1.1546x; 1.1497x over previous
import functools

import jax
import jax.numpy as jnp
from jax import lax
from jax.experimental import pallas as pl
from jax.experimental.pallas import tpu as pltpu
from jax.experimental.pallas import tpu_sc as plsc

N = 10000
K = 16
CIN = 128
COUT = 32
M = 4

NW = 32
N_PAD = 10240
C = 8
NBUF = 4
PER_W0 = 320
PER_W1 = 320
PER_W_MAX = PER_W0
CORE1_BASE = PER_W0 * 16


def _mm_body(x_ref, w_ref, y_ref):
    y_ref[...] = jnp.dot(x_ref[...], w_ref[...],
                         preferred_element_type=jnp.float32)


def _tc_matmul(x2, wcat):
    blk = 2048
    return pl.pallas_call(
        _mm_body,
        grid=(N_PAD // blk,),
        in_specs=[pl.BlockSpec((blk, CIN), lambda i: (i, 0)),
                  pl.BlockSpec((CIN, 256), lambda i: (0, 0)),],
        out_specs=pl.BlockSpec((blk, 256), lambda i: (i, 0)),
        out_shape=jax.ShapeDtypeStruct((N_PAD, 256), jnp.float32),
    )(x2, wcat)


def _sc_body(wx_hbm, uxf_hbm, adjf_hbm, c_hbm, b_hbm, out_hbm,
             idx_list, adj_all, wrows_list, uxf_v, out_all,
             cvec, bvec, sems):
    cid = lax.axis_index("c")
    sid = lax.axis_index("s")
    pltpu.sync_copy(c_hbm, cvec)
    pltpu.sync_copy(b_hbm, bvec)

    pltpu.sync_copy(uxf_hbm, uxf_v)
    cv = cvec[...]
    cs = [cv[m] for m in range(M)]
    b_lo = bvec[pl.ds(0, 16)]
    b_hi = bvec[pl.ds(16, 16)]

    def worker(base_w, per_w):
        n_pairs = per_w // C // 2
        pltpu.sync_copy(adjf_hbm.at[pl.ds(base_w * K, per_w * K)],
                        adj_all.at[pl.ds(0, per_w * K)])

        def build_idx(idx_ref, ci):
            for cc in range(C):
                a = adj_all[pl.ds((ci * C + cc) * K, K)]
                idx_ref[pl.ds(cc * K, K)] = jnp.maximum(a - 1, 0)

        def compute_chunk(wrows, ci):
            def node_body(cc, _):
                loc = ci * C + cc
                a = adj_all[pl.ds(loc * K, K)]
                valid = a > 0
                cnt = jnp.zeros((16,), jnp.float32) + jnp.sum(
                    jnp.where(valid, 1.0, 0.0))
                invc = jnp.where(cnt > 0.0, 1.0 / cnt, 0.0)
                idx0 = jnp.maximum(a - 1, 0)
                base4 = idx0 * M
                own = (base_w + loc) * M
                ps = []
                for m in range(M):
                    uxg = plsc.load_gather(uxf_v, [base4 + m])
                    uo = plsc.load_gather(
                        uxf_v, [jnp.full((16,), m, jnp.int32) + own])
                    ps.append((uo + cs[m]) - uxg)
                pmax = jnp.maximum(jnp.maximum(ps[0], ps[1]),
                                   jnp.maximum(ps[2], ps[3]))
                es = [jnp.exp(p - pmax) for p in ps]
                ssum = (es[0] + es[1]) + (es[2] + es[3])
                scale = invc / ssum
                wms = [jnp.where(valid, e * scale, 0.0) for e in es]
                acc_lo = b_lo
                acc_hi = b_hi
                for k in range(K):
                    j = cc * K + k
                    for m in range(M):
                        w = wms[m][k]
                        acc_lo = acc_lo + w * wrows[j, pl.ds(32 * m, 16)]
                        acc_hi = acc_hi + w * wrows[j, pl.ds(32 * m + 16, 16)]
                out_all[pl.ds(loc * COUT, 16)] = acc_lo
                out_all[pl.ds(loc * COUT + 16, 16)] = acc_hi
                return 0

            lax.fori_loop(0, C, node_body, 0)

        n_chunks = per_w // C
        n_outer = n_chunks // NBUF

        def fire(b, ci):
            build_idx(idx_list[b], ci)
            pltpu.async_copy(wx_hbm.at[idx_list[b]], wrows_list[b], sems[b])

        def ring_body(g, _):
            for b in range(NBUF):
                ci = g * NBUF + b
                pltpu.make_async_copy(
                    wx_hbm.at[idx_list[b]], wrows_list[b], sems[b]).wait()
                compute_chunk(wrows_list[b], ci)

                @pl.when(ci + NBUF < n_chunks)
                def _():
                    fire(b, ci + NBUF)

            return 0

        for b in range(NBUF):
            fire(b, b)
        lax.fori_loop(0, n_outer, ring_body, 0)
        pltpu.sync_copy(out_all.at[pl.ds(0, per_w * COUT)],
                        out_hbm.at[pl.ds(base_w * COUT, per_w * COUT)])

    @pl.when(cid == 0)
    def _():
        worker(sid * PER_W0, PER_W0)

    @pl.when(cid == 1)
    def _():
        worker(CORE1_BASE + sid * PER_W1, PER_W1)


_sc_kernel = functools.partial(
    pl.kernel,
    mesh=plsc.VectorSubcoreMesh(core_axis_name="c", subcore_axis_name="s"),
    compiler_params=pltpu.CompilerParams(needs_layout_passes=False),
    out_type=jax.ShapeDtypeStruct((N_PAD * COUT,), jnp.float32),
    scratch_types=[
        [pltpu.VMEM((C * K,), jnp.int32) for _ in range(NBUF)],
        pltpu.VMEM((PER_W_MAX * K,), jnp.int32),
        [pltpu.VMEM((C * K, CIN), jnp.float32) for _ in range(NBUF)],
        pltpu.VMEM((N_PAD * M,), jnp.float32),
        pltpu.VMEM((PER_W_MAX * COUT,), jnp.float32),
        pltpu.VMEM((16,), jnp.float32),
        pltpu.VMEM((COUT,), jnp.float32),
        [pltpu.SemaphoreType.DMA for _ in range(NBUF)],
    ],
)(_sc_body)


def kernel(x, adj, W, b, u, c):
    x2 = x[0]
    x2p = jnp.pad(x2, ((0, N_PAD - N), (0, 0)))
    Wr = W.reshape(M * COUT, CIN)
    wcat = jnp.concatenate(
        [Wr.T, u.T, jnp.zeros((CIN, 256 - M * COUT - M), jnp.float32)],
        axis=1)
    y = _tc_matmul(x2p, wcat)
    wx = y[:, :M * COUT]
    uxf = y[:, M * COUT:M * COUT + M].reshape(-1)
    adjf = jnp.pad(adj, ((0, N_PAD - N), (0, 0))).reshape(-1)
    c_pad = jnp.pad(c, (0, 16 - M))
    out = _sc_kernel(wx, uxf, adjf, c_pad, b)
    return out[:N * COUT].reshape(1, N, COUT)


# speedup vs baseline: 1.7438x; 1.5103x over previous
import functools

import jax
import jax.numpy as jnp
from jax import lax
from jax.experimental import pallas as pl
from jax.experimental.pallas import tpu as pltpu
from jax.experimental.pallas import tpu_sc as plsc

N = 10000
K = 16
CIN = 128
COUT = 32
M = 4

NW = 32
N_PAD = 10240
C = 8
NBUF = 4
PER_W0 = 320
PER_W1 = 320
PER_W_MAX = PER_W0
CORE1_BASE = PER_W0 * 16


def _mm_body(x_ref, w_ref, u_ref, y_ref, z_ref):
    xv = x_ref[...]
    y_ref[...] = jnp.dot(xv, w_ref[...],
                         preferred_element_type=jnp.float32
                         ).astype(jnp.bfloat16)
    z_ref[...] = jnp.dot(xv, u_ref[...], preferred_element_type=jnp.float32)


def _tc_matmul(x2, wperm, ut):
    blk = 2048
    return pl.pallas_call(
        _mm_body,
        grid=(N_PAD // blk,),
        in_specs=[pl.BlockSpec((blk, CIN), lambda i: (i, 0)),
                  pl.BlockSpec((CIN, M * COUT), lambda i: (0, 0)),
                  pl.BlockSpec((CIN, M), lambda i: (0, 0))],
        out_specs=[pl.BlockSpec((blk, M * COUT), lambda i: (i, 0)),
                   pl.BlockSpec((blk, M), lambda i: (i, 0))],
        out_shape=[jax.ShapeDtypeStruct((N_PAD, M * COUT), jnp.bfloat16),
                   jax.ShapeDtypeStruct((N_PAD, M), jnp.float32)],
    )(x2, wperm, ut)


def _sc_body(wx_hbm, uxf_hbm, adjf_hbm, c_hbm, b_hbm, out_hbm,
             idx_list, adj_all, wrows_list, uxf_v, out_all,
             cvec, bvec, sems):
    cid = lax.axis_index("c")
    sid = lax.axis_index("s")
    pltpu.sync_copy(c_hbm, cvec)
    pltpu.sync_copy(b_hbm, bvec)

    pltpu.sync_copy(uxf_hbm, uxf_v)
    cv = cvec[...]
    cs = [cv[m] for m in range(M)]
    b_lo = bvec[pl.ds(0, 16)]
    b_hi = bvec[pl.ds(16, 16)]

    def worker(base_w, per_w):
        n_pairs = per_w // C // 2
        pltpu.sync_copy(adjf_hbm.at[pl.ds(base_w * K, per_w * K)],
                        adj_all.at[pl.ds(0, per_w * K)])

        def build_idx(idx_ref, ci):
            for cc in range(C):
                a = adj_all[pl.ds((ci * C + cc) * K, K)]
                idx_ref[pl.ds(cc * K, K)] = jnp.maximum(a - 1, 0)

        def compute_chunk(wrows, ci):
            def node_body(cc, _):
                loc = ci * C + cc
                a = adj_all[pl.ds(loc * K, K)]
                valid = a > 0
                cnt = jnp.zeros((16,), jnp.float32) + jnp.sum(
                    jnp.where(valid, 1.0, 0.0))
                invc = jnp.where(cnt > 0.0, 1.0 / cnt, 0.0)
                idx0 = jnp.maximum(a - 1, 0)
                base4 = idx0 * M
                own = (base_w + loc) * M
                ps = []
                for m in range(M):
                    uxg = plsc.load_gather(uxf_v, [base4 + m])
                    uo = plsc.load_gather(
                        uxf_v, [jnp.full((16,), m, jnp.int32) + own])
                    ps.append((uo + cs[m]) - uxg)
                pmax = jnp.maximum(jnp.maximum(ps[0], ps[1]),
                                   jnp.maximum(ps[2], ps[3]))
                es = [jnp.exp(p - pmax) for p in ps]
                ssum = (es[0] + es[1]) + (es[2] + es[3])
                scale = invc / ssum
                wms = [jnp.where(valid, e * scale, 0.0) for e in es]
                acc_lo = b_lo
                acc_hi = b_hi
                for k in range(K):
                    j = cc * K + k
                    for m in range(M):
                        w = wms[m][k]
                        ab = wrows[j, pl.ds(32 * m, 32)]
                        lo, hi = plsc.unpack(
                            ab, format=plsc.PackFormat.INTERLEAVED)
                        acc_lo = acc_lo + w * lo
                        acc_hi = acc_hi + w * hi
                out_all[pl.ds(loc * COUT, 16)] = acc_lo
                out_all[pl.ds(loc * COUT + 16, 16)] = acc_hi
                return 0

            lax.fori_loop(0, C, node_body, 0)

        n_chunks = per_w // C
        n_outer = n_chunks // NBUF

        def fire(b, ci):
            build_idx(idx_list[b], ci)
            pltpu.async_copy(wx_hbm.at[idx_list[b]], wrows_list[b], sems[b])

        def ring_body(g, _):
            for b in range(NBUF):
                ci = g * NBUF + b
                pltpu.make_async_copy(
                    wx_hbm.at[idx_list[b]], wrows_list[b], sems[b]).wait()
                compute_chunk(wrows_list[b], ci)

                @pl.when(ci + NBUF < n_chunks)
                def _():
                    fire(b, ci + NBUF)

            return 0

        for b in range(NBUF):
            fire(b, b)
        lax.fori_loop(0, n_outer, ring_body, 0)
        pltpu.sync_copy(out_all.at[pl.ds(0, per_w * COUT)],
                        out_hbm.at[pl.ds(base_w * COUT, per_w * COUT)])

    @pl.when(cid == 0)
    def _():
        worker(sid * PER_W0, PER_W0)

    @pl.when(cid == 1)
    def _():
        worker(CORE1_BASE + sid * PER_W1, PER_W1)


_sc_kernel = functools.partial(
    pl.kernel,
    mesh=plsc.VectorSubcoreMesh(core_axis_name="c", subcore_axis_name="s"),
    compiler_params=pltpu.CompilerParams(needs_layout_passes=False,
                                         use_tc_tiling_on_sc=False),
    out_type=jax.ShapeDtypeStruct((N_PAD * COUT,), jnp.float32),
    scratch_types=[
        [pltpu.VMEM((C * K,), jnp.int32) for _ in range(NBUF)],
        pltpu.VMEM((PER_W_MAX * K,), jnp.int32),
        [pltpu.VMEM((C * K, CIN), jnp.bfloat16) for _ in range(NBUF)],
        pltpu.VMEM((N_PAD * M,), jnp.float32),
        pltpu.VMEM((PER_W_MAX * COUT,), jnp.float32),
        pltpu.VMEM((16,), jnp.float32),
        pltpu.VMEM((COUT,), jnp.float32),
        [pltpu.SemaphoreType.DMA for _ in range(NBUF)],
    ],
)(_sc_body)


def kernel(x, adj, W, b, u, c):
    x2 = x[0]
    x2p = jnp.pad(x2, ((0, N_PAD - N), (0, 0)))
    Wr = W.reshape(M * COUT, CIN)
    perm = []
    for m in range(M):
        for i in range(16):
            perm.append(32 * m + i)
            perm.append(32 * m + 16 + i)
    perm = jnp.array(perm, dtype=jnp.int32)
    wx, z = _tc_matmul(x2p, Wr.T[:, perm], u.T)
    uxf = z.reshape(-1)
    adjf = jnp.pad(adj, ((0, N_PAD - N), (0, 0))).reshape(-1)
    c_pad = jnp.pad(c, (0, 16 - M))
    out = _sc_kernel(wx, uxf, adjf, c_pad, b)
    return out[:N * COUT].reshape(1, N, COUT)


# speedup vs baseline: 1.8367x; 1.0533x over previous
import functools

import jax
import jax.numpy as jnp
from jax import lax
from jax.experimental import pallas as pl
from jax.experimental.pallas import tpu as pltpu
from jax.experimental.pallas import tpu_sc as plsc

N = 10000
K = 16
CIN = 128
COUT = 32
M = 4

NW = 32
N_PAD = 10240
C = 8
NBUF = 4
PER_W0 = 448
PER_W1 = 192
PER_W_MAX = PER_W0
CORE1_BASE = PER_W0 * 16


def _mm_body(x_ref, w_ref, u_ref, y_ref, z_ref):
    xv = x_ref[...]
    y_ref[...] = jnp.dot(xv, w_ref[...],
                         preferred_element_type=jnp.float32
                         ).astype(jnp.bfloat16)
    z_ref[...] = jnp.dot(xv, u_ref[...], preferred_element_type=jnp.float32)


def _tc_matmul(x2, wperm, ut):
    blk = 2048
    return pl.pallas_call(
        _mm_body,
        grid=(N_PAD // blk,),
        in_specs=[pl.BlockSpec((blk, CIN), lambda i: (i, 0)),
                  pl.BlockSpec((CIN, M * COUT), lambda i: (0, 0)),
                  pl.BlockSpec((CIN, M), lambda i: (0, 0))],
        out_specs=[pl.BlockSpec((blk, M * COUT), lambda i: (i, 0)),
                   pl.BlockSpec((blk, M), lambda i: (i, 0))],
        out_shape=[jax.ShapeDtypeStruct((N_PAD, M * COUT), jnp.bfloat16),
                   jax.ShapeDtypeStruct((N_PAD, M), jnp.float32)],
    )(x2, wperm, ut)


def _sc_body(wx_hbm, uxf_hbm, adjf_hbm, c_hbm, b_hbm, out_hbm,
             idx_list, adj_all, wrows_list, uxf_v, out_all,
             cvec, bvec, sems):
    cid = lax.axis_index("c")
    sid = lax.axis_index("s")
    pltpu.sync_copy(c_hbm, cvec)
    pltpu.sync_copy(b_hbm, bvec)

    pltpu.sync_copy(uxf_hbm, uxf_v)
    cv = cvec[...]
    cs = [cv[m] for m in range(M)]
    b_lo = bvec[pl.ds(0, 16)]
    b_hi = bvec[pl.ds(16, 16)]

    def worker(base_w, per_w):
        n_pairs = per_w // C // 2
        pltpu.sync_copy(adjf_hbm.at[pl.ds(base_w * K, per_w * K)],
                        adj_all.at[pl.ds(0, per_w * K)])

        def build_idx(idx_ref, ci):
            for cc in range(C):
                a = adj_all[pl.ds((ci * C + cc) * K, K)]
                idx_ref[pl.ds(cc * K, K)] = jnp.maximum(a - 1, 0)

        def compute_chunk(wrows, ci):
            def node_body(cc, _):
                loc = ci * C + cc
                a = adj_all[pl.ds(loc * K, K)]
                valid = a > 0
                cnt = jnp.zeros((16,), jnp.float32) + jnp.sum(
                    jnp.where(valid, 1.0, 0.0))
                invc = jnp.where(cnt > 0.0, 1.0 / cnt, 0.0)
                idx0 = jnp.maximum(a - 1, 0)
                base4 = idx0 * M
                own = (base_w + loc) * M
                ps = []
                for m in range(M):
                    uxg = plsc.load_gather(uxf_v, [base4 + m])
                    uo = plsc.load_gather(
                        uxf_v, [jnp.full((16,), m, jnp.int32) + own])
                    ps.append((uo + cs[m]) - uxg)
                pmax = jnp.maximum(jnp.maximum(ps[0], ps[1]),
                                   jnp.maximum(ps[2], ps[3]))
                es = [jnp.exp(p - pmax) for p in ps]
                ssum = (es[0] + es[1]) + (es[2] + es[3])
                scale = invc / ssum
                wms = [jnp.where(valid, e * scale, 0.0) for e in es]
                acc_lo = b_lo
                acc_hi = b_hi
                for k in range(K):
                    j = cc * K + k
                    for m in range(M):
                        w = wms[m][k]
                        ab = wrows[j, pl.ds(32 * m, 32)]
                        lo, hi = plsc.unpack(
                            ab, format=plsc.PackFormat.INTERLEAVED)
                        acc_lo = acc_lo + w * lo
                        acc_hi = acc_hi + w * hi
                out_all[pl.ds(loc * COUT, 16)] = acc_lo
                out_all[pl.ds(loc * COUT + 16, 16)] = acc_hi
                return 0

            lax.fori_loop(0, C, node_body, 0)

        n_chunks = per_w // C
        n_outer = n_chunks // NBUF

        def fire(b, ci):
            build_idx(idx_list[b], ci)
            pltpu.async_copy(wx_hbm.at[idx_list[b]], wrows_list[b], sems[b])

        def ring_body(g, _):
            for b in range(NBUF):
                ci = g * NBUF + b
                pltpu.make_async_copy(
                    wx_hbm.at[idx_list[b]], wrows_list[b], sems[b]).wait()
                compute_chunk(wrows_list[b], ci)

                @pl.when(ci + NBUF < n_chunks)
                def _():
                    fire(b, ci + NBUF)

            return 0

        for b in range(NBUF):
            fire(b, b)
        lax.fori_loop(0, n_outer, ring_body, 0)
        pltpu.sync_copy(out_all.at[pl.ds(0, per_w * COUT)],
                        out_hbm.at[pl.ds(base_w * COUT, per_w * COUT)])

    @pl.when(cid == 0)
    def _():
        worker(sid * PER_W0, PER_W0)

    @pl.when(cid == 1)
    def _():
        worker(CORE1_BASE + sid * PER_W1, PER_W1)


_sc_kernel = functools.partial(
    pl.kernel,
    mesh=plsc.VectorSubcoreMesh(core_axis_name="c", subcore_axis_name="s"),
    compiler_params=pltpu.CompilerParams(needs_layout_passes=False,
                                         use_tc_tiling_on_sc=False),
    out_type=jax.ShapeDtypeStruct((N_PAD * COUT,), jnp.float32),
    scratch_types=[
        [pltpu.VMEM((C * K,), jnp.int32) for _ in range(NBUF)],
        pltpu.VMEM((PER_W_MAX * K,), jnp.int32),
        [pltpu.VMEM((C * K, CIN), jnp.bfloat16) for _ in range(NBUF)],
        pltpu.VMEM((N_PAD * M,), jnp.float32),
        pltpu.VMEM((PER_W_MAX * COUT,), jnp.float32),
        pltpu.VMEM((16,), jnp.float32),
        pltpu.VMEM((COUT,), jnp.float32),
        [pltpu.SemaphoreType.DMA for _ in range(NBUF)],
    ],
)(_sc_body)


def kernel(x, adj, W, b, u, c):
    x2 = x[0]
    x2p = jnp.pad(x2, ((0, N_PAD - N), (0, 0)))
    Wr = W.reshape(M * COUT, CIN)
    perm = []
    for m in range(M):
        for i in range(16):
            perm.append(32 * m + i)
            perm.append(32 * m + 16 + i)
    perm = jnp.array(perm, dtype=jnp.int32)
    wx, z = _tc_matmul(x2p, Wr.T[:, perm], u.T)
    uxf = z.reshape(-1)
    adjf = jnp.pad(adj, ((0, N_PAD - N), (0, 0))).reshape(-1)
    c_pad = jnp.pad(c, (0, 16 - M))
    out = _sc_kernel(wx, uxf, adjf, c_pad, b)
    return out[:N * COUT].reshape(1, N, COUT)


# speedup vs baseline: 1.8672x; 1.0166x over previous
import functools

import jax
import jax.numpy as jnp
from jax import lax
from jax.experimental import pallas as pl
from jax.experimental.pallas import tpu as pltpu
from jax.experimental.pallas import tpu_sc as plsc

N = 10000
K = 16
CIN = 128
COUT = 32
M = 4

NW = 32
N_PAD = 10240
C = 8
NBUF = 4
PER_W0 = 448
PER_W1 = 192
PER_W_MAX = PER_W0
CORE1_BASE = PER_W0 * 16


def _mm_body(x_ref, w_ref, u_ref, y_ref, z_ref):
    xv = x_ref[...]
    y_ref[...] = jnp.dot(xv, w_ref[...],
                         preferred_element_type=jnp.float32
                         ).astype(jnp.bfloat16)
    z_ref[...] = jnp.dot(xv, u_ref[...], preferred_element_type=jnp.float32)


def _tc_matmul(x2, wperm, ut):
    blk = 2048
    return pl.pallas_call(
        _mm_body,
        grid=(N_PAD // blk,),
        in_specs=[pl.BlockSpec((blk, CIN), lambda i: (i, 0)),
                  pl.BlockSpec((CIN, M * COUT), lambda i: (0, 0)),
                  pl.BlockSpec((CIN, M), lambda i: (0, 0))],
        out_specs=[pl.BlockSpec((blk, M * COUT), lambda i: (i, 0)),
                   pl.BlockSpec((blk, M), lambda i: (i, 0))],
        out_shape=[jax.ShapeDtypeStruct((N_PAD, M * COUT), jnp.bfloat16),
                   jax.ShapeDtypeStruct((N_PAD, M), jnp.float32)],
    )(x2, wperm, ut)


def _sc_body(wx_hbm, uxf_hbm, adjf_hbm, c_hbm, b_hbm, out_hbm,
             idx_list, adj_all, wrows_list, uxf_v, ux_sh, out_all,
             cvec, bvec, sems):
    cid = lax.axis_index("c")
    sid = lax.axis_index("s")
    pltpu.sync_copy(c_hbm, cvec)
    pltpu.sync_copy(b_hbm, bvec)

    @pl.when(sid == 0)
    def _():
        pltpu.sync_copy(uxf_hbm, ux_sh)

    plsc.subcore_barrier()
    pltpu.sync_copy(ux_sh, uxf_v)
    cv = cvec[...]
    cs = [cv[m] for m in range(M)]
    b_lo = bvec[pl.ds(0, 16)]
    b_hi = bvec[pl.ds(16, 16)]

    def worker(base_w, per_w):
        n_pairs = per_w // C // 2
        pltpu.sync_copy(adjf_hbm.at[pl.ds(base_w * K, per_w * K)],
                        adj_all.at[pl.ds(0, per_w * K)])

        def build_idx(idx_ref, ci):
            for cc in range(C):
                a = adj_all[pl.ds((ci * C + cc) * K, K)]
                idx_ref[pl.ds(cc * K, K)] = jnp.maximum(a - 1, 0)

        def compute_chunk(wrows, ci):
            def node_body(cc, _):
                loc = ci * C + cc
                a = adj_all[pl.ds(loc * K, K)]
                valid = a > 0
                cnt = jnp.zeros((16,), jnp.float32) + jnp.sum(
                    jnp.where(valid, 1.0, 0.0))
                invc = jnp.where(cnt > 0.0, 1.0 / cnt, 0.0)
                idx0 = jnp.maximum(a - 1, 0)
                base4 = idx0 * M
                own = (base_w + loc) * M
                ps = []
                for m in range(M):
                    uxg = plsc.load_gather(uxf_v, [base4 + m])
                    uo = plsc.load_gather(
                        uxf_v, [jnp.full((16,), m, jnp.int32) + own])
                    ps.append((uo + cs[m]) - uxg)
                pmax = jnp.maximum(jnp.maximum(ps[0], ps[1]),
                                   jnp.maximum(ps[2], ps[3]))
                es = [jnp.exp(p - pmax) for p in ps]
                ssum = (es[0] + es[1]) + (es[2] + es[3])
                scale = invc / ssum
                wms = [jnp.where(valid, e * scale, 0.0) for e in es]
                acc_lo = b_lo
                acc_hi = b_hi
                for k in range(K):
                    j = cc * K + k
                    for m in range(M):
                        w = wms[m][k]
                        ab = wrows[j, pl.ds(32 * m, 32)]
                        lo, hi = plsc.unpack(
                            ab, format=plsc.PackFormat.INTERLEAVED)
                        acc_lo = acc_lo + w * lo
                        acc_hi = acc_hi + w * hi
                out_all[pl.ds(loc * COUT, 16)] = acc_lo
                out_all[pl.ds(loc * COUT + 16, 16)] = acc_hi
                return 0

            lax.fori_loop(0, C, node_body, 0)

        n_chunks = per_w // C
        n_outer = n_chunks // NBUF

        def fire(b, ci):
            build_idx(idx_list[b], ci)
            pltpu.async_copy(wx_hbm.at[idx_list[b]], wrows_list[b], sems[b])

        def ring_body(g, _):
            for b in range(NBUF):
                ci = g * NBUF + b
                pltpu.make_async_copy(
                    wx_hbm.at[idx_list[b]], wrows_list[b], sems[b]).wait()
                compute_chunk(wrows_list[b], ci)

                @pl.when(ci + NBUF < n_chunks)
                def _():
                    fire(b, ci + NBUF)

            return 0

        for b in range(NBUF):
            fire(b, b)
        lax.fori_loop(0, n_outer, ring_body, 0)
        pltpu.sync_copy(out_all.at[pl.ds(0, per_w * COUT)],
                        out_hbm.at[pl.ds(base_w * COUT, per_w * COUT)])

    @pl.when(cid == 0)
    def _():
        worker(sid * PER_W0, PER_W0)

    @pl.when(cid == 1)
    def _():
        worker(CORE1_BASE + sid * PER_W1, PER_W1)


_sc_kernel = functools.partial(
    pl.kernel,
    mesh=plsc.VectorSubcoreMesh(core_axis_name="c", subcore_axis_name="s"),
    compiler_params=pltpu.CompilerParams(needs_layout_passes=False,
                                         use_tc_tiling_on_sc=False),
    out_type=jax.ShapeDtypeStruct((N_PAD * COUT,), jnp.float32),
    scratch_types=[
        [pltpu.VMEM((C * K,), jnp.int32) for _ in range(NBUF)],
        pltpu.VMEM((PER_W_MAX * K,), jnp.int32),
        [pltpu.VMEM((C * K, CIN), jnp.bfloat16) for _ in range(NBUF)],
        pltpu.VMEM((N_PAD * M,), jnp.float32),
        pltpu.VMEM_SHARED((N_PAD * M,), jnp.float32),
        pltpu.VMEM((PER_W_MAX * COUT,), jnp.float32),
        pltpu.VMEM((16,), jnp.float32),
        pltpu.VMEM((COUT,), jnp.float32),
        [pltpu.SemaphoreType.DMA for _ in range(NBUF)],
    ],
)(_sc_body)


def kernel(x, adj, W, b, u, c):
    x2 = x[0]
    x2p = jnp.pad(x2, ((0, N_PAD - N), (0, 0)))
    Wr = W.reshape(M * COUT, CIN)
    perm = []
    for m in range(M):
        for i in range(16):
            perm.append(32 * m + i)
            perm.append(32 * m + 16 + i)
    perm = jnp.array(perm, dtype=jnp.int32)
    wx, z = _tc_matmul(x2p, Wr.T[:, perm], u.T)
    uxf = z.reshape(-1)
    adjf = jnp.pad(adj, ((0, N_PAD - N), (0, 0))).reshape(-1)
    c_pad = jnp.pad(c, (0, 16 - M))
    out = _sc_kernel(wx, uxf, adjf, c_pad, b)
    return out[:N * COUT].reshape(1, N, COUT)


# speedup vs baseline: 1.8863x; 1.0102x over previous
import functools

import jax
import jax.numpy as jnp
from jax import lax
from jax.experimental import pallas as pl
from jax.experimental.pallas import tpu as pltpu
from jax.experimental.pallas import tpu_sc as plsc

N = 10000
K = 16
CIN = 128
COUT = 32
M = 4

NW = 32
N_PAD = 10240
C = 8
NBUF = 2
PER_W0 = 448
PER_W1 = 192
PER_W_MAX = PER_W0
CORE1_BASE = PER_W0 * 16


def _mm_body(x_ref, w_ref, u_ref, y_ref, z_ref):
    xv = x_ref[...]
    y_ref[...] = jnp.dot(xv, w_ref[...],
                         preferred_element_type=jnp.float32
                         ).astype(jnp.bfloat16)
    z_ref[...] = jnp.dot(xv, u_ref[...], preferred_element_type=jnp.float32)


def _tc_matmul(x2, wperm, ut):
    blk = 2048
    return pl.pallas_call(
        _mm_body,
        grid=(N_PAD // blk,),
        in_specs=[pl.BlockSpec((blk, CIN), lambda i: (i, 0)),
                  pl.BlockSpec((CIN, M * COUT), lambda i: (0, 0)),
                  pl.BlockSpec((CIN, M), lambda i: (0, 0))],
        out_specs=[pl.BlockSpec((blk, M * COUT), lambda i: (i, 0)),
                   pl.BlockSpec((blk, M), lambda i: (i, 0))],
        out_shape=[jax.ShapeDtypeStruct((N_PAD, M * COUT), jnp.bfloat16),
                   jax.ShapeDtypeStruct((N_PAD, M), jnp.float32)],
    )(x2, wperm, ut)


def _sc_body(wx_hbm, uxf_hbm, adjf_hbm, c_hbm, b_hbm, out_hbm,
             idx_list, adj_all, wrows_list, uxf_v, ux_sh, out_all,
             cvec, bvec, sems):
    cid = lax.axis_index("c")
    sid = lax.axis_index("s")
    pltpu.sync_copy(c_hbm, cvec)
    pltpu.sync_copy(b_hbm, bvec)

    @pl.when(sid == 0)
    def _():
        pltpu.sync_copy(uxf_hbm, ux_sh)

    plsc.subcore_barrier()
    pltpu.sync_copy(ux_sh, uxf_v)
    cv = cvec[...]
    cs = [cv[m] for m in range(M)]
    b_lo = bvec[pl.ds(0, 16)]
    b_hi = bvec[pl.ds(16, 16)]

    def worker(base_w, per_w):
        n_pairs = per_w // C // 2
        pltpu.sync_copy(adjf_hbm.at[pl.ds(base_w * K, per_w * K)],
                        adj_all.at[pl.ds(0, per_w * K)])

        def build_idx(idx_ref, ci):
            for cc in range(C):
                a = adj_all[pl.ds((ci * C + cc) * K, K)]
                idx_ref[pl.ds(cc * K, K)] = jnp.maximum(a - 1, 0)

        def compute_chunk(wrows, ci):
            def node_body(cc, _):
                loc = ci * C + cc
                a = adj_all[pl.ds(loc * K, K)]
                valid = a > 0
                cnt = jnp.zeros((16,), jnp.float32) + jnp.sum(
                    jnp.where(valid, 1.0, 0.0))
                invc = jnp.where(cnt > 0.0, 1.0 / cnt, 0.0)
                idx0 = jnp.maximum(a - 1, 0)
                base4 = idx0 * M
                own = (base_w + loc) * M
                ps = []
                for m in range(M):
                    uxg = plsc.load_gather(uxf_v, [base4 + m])
                    uo = plsc.load_gather(
                        uxf_v, [jnp.full((16,), m, jnp.int32) + own])
                    ps.append((uo + cs[m]) - uxg)
                pmax = jnp.maximum(jnp.maximum(ps[0], ps[1]),
                                   jnp.maximum(ps[2], ps[3]))
                es = [jnp.exp(p - pmax) for p in ps]
                ssum = (es[0] + es[1]) + (es[2] + es[3])
                scale = invc / ssum
                wms = [jnp.where(valid, e * scale, 0.0) for e in es]
                acc_lo = b_lo
                acc_hi = b_hi
                for k in range(K):
                    j = cc * K + k
                    for m in range(M):
                        w = wms[m][k]
                        ab = wrows[j, pl.ds(32 * m, 32)]
                        lo, hi = plsc.unpack(
                            ab, format=plsc.PackFormat.INTERLEAVED)
                        acc_lo = acc_lo + w * lo
                        acc_hi = acc_hi + w * hi
                out_all[pl.ds(loc * COUT, 16)] = acc_lo
                out_all[pl.ds(loc * COUT + 16, 16)] = acc_hi
                return 0

            lax.fori_loop(0, C, node_body, 0)

        n_chunks = per_w // C
        n_outer = n_chunks // NBUF

        def fire(b, ci):
            build_idx(idx_list[b], ci)
            pltpu.async_copy(wx_hbm.at[idx_list[b]], wrows_list[b], sems[b])

        def ring_body(g, _):
            for b in range(NBUF):
                ci = g * NBUF + b
                pltpu.make_async_copy(
                    wx_hbm.at[idx_list[b]], wrows_list[b], sems[b]).wait()
                compute_chunk(wrows_list[b], ci)

                @pl.when(ci + NBUF < n_chunks)
                def _():
                    fire(b, ci + NBUF)

            return 0

        for b in range(NBUF):
            fire(b, b)
        lax.fori_loop(0, n_outer, ring_body, 0)
        pltpu.sync_copy(out_all.at[pl.ds(0, per_w * COUT)],
                        out_hbm.at[pl.ds(base_w * COUT, per_w * COUT)])

    @pl.when(cid == 0)
    def _():
        worker(sid * PER_W0, PER_W0)

    @pl.when(cid == 1)
    def _():
        worker(CORE1_BASE + sid * PER_W1, PER_W1)


_sc_kernel = functools.partial(
    pl.kernel,
    mesh=plsc.VectorSubcoreMesh(core_axis_name="c", subcore_axis_name="s"),
    compiler_params=pltpu.CompilerParams(needs_layout_passes=False,
                                         use_tc_tiling_on_sc=False),
    out_type=jax.ShapeDtypeStruct((N_PAD * COUT,), jnp.float32),
    scratch_types=[
        [pltpu.VMEM((C * K,), jnp.int32) for _ in range(NBUF)],
        pltpu.VMEM((PER_W_MAX * K,), jnp.int32),
        [pltpu.VMEM((C * K, CIN), jnp.bfloat16) for _ in range(NBUF)],
        pltpu.VMEM((N_PAD * M,), jnp.float32),
        pltpu.VMEM_SHARED((N_PAD * M,), jnp.float32),
        pltpu.VMEM((PER_W_MAX * COUT,), jnp.float32),
        pltpu.VMEM((16,), jnp.float32),
        pltpu.VMEM((COUT,), jnp.float32),
        [pltpu.SemaphoreType.DMA for _ in range(NBUF)],
    ],
)(_sc_body)


def kernel(x, adj, W, b, u, c):
    x2 = x[0]
    x2p = jnp.pad(x2, ((0, N_PAD - N), (0, 0)))
    Wr = W.reshape(M * COUT, CIN)
    perm = []
    for m in range(M):
        for i in range(16):
            perm.append(32 * m + i)
            perm.append(32 * m + 16 + i)
    perm = jnp.array(perm, dtype=jnp.int32)
    wx, z = _tc_matmul(x2p, Wr.T[:, perm], u.T)
    uxf = z.reshape(-1)
    adjf = jnp.pad(adj, ((0, N_PAD - N), (0, 0))).reshape(-1)
    c_pad = jnp.pad(c, (0, 16 - M))
    out = _sc_kernel(wx, uxf, adjf, c_pad, b)
    return out[:N * COUT].reshape(1, N, COUT)


# speedup vs baseline: 1.8961x; 1.0052x over previous
import functools

import jax
import jax.numpy as jnp
from jax import lax
from jax.experimental import pallas as pl
from jax.experimental.pallas import tpu as pltpu
from jax.experimental.pallas import tpu_sc as plsc

N = 10000
K = 16
CIN = 128
COUT = 32
M = 4

NW = 32
N_PAD = 10240
C = 8
NBUF = 2
PER_W0 = 496
PER_W1 = 144
PER_W_MAX = PER_W0
CORE1_BASE = PER_W0 * 16


def _mm_body(x_ref, w_ref, u_ref, y_ref, z_ref):
    xv = x_ref[...]
    y_ref[...] = jnp.dot(xv, w_ref[...],
                         preferred_element_type=jnp.float32
                         ).astype(jnp.bfloat16)
    z_ref[...] = jnp.dot(xv, u_ref[...], preferred_element_type=jnp.float32)


def _tc_matmul(x2, wperm, ut):
    blk = 2048
    return pl.pallas_call(
        _mm_body,
        grid=(N_PAD // blk,),
        in_specs=[pl.BlockSpec((blk, CIN), lambda i: (i, 0)),
                  pl.BlockSpec((CIN, M * COUT), lambda i: (0, 0)),
                  pl.BlockSpec((CIN, M), lambda i: (0, 0))],
        out_specs=[pl.BlockSpec((blk, M * COUT), lambda i: (i, 0)),
                   pl.BlockSpec((blk, M), lambda i: (i, 0))],
        out_shape=[jax.ShapeDtypeStruct((N_PAD, M * COUT), jnp.bfloat16),
                   jax.ShapeDtypeStruct((N_PAD, M), jnp.float32)],
    )(x2, wperm, ut)


def _sc_body(wx_hbm, uxf_hbm, adjf_hbm, c_hbm, b_hbm, out_hbm,
             idx_list, adj_all, wrows_list, uxf_v, ux_sh, out_all,
             cvec, bvec, sems):
    cid = lax.axis_index("c")
    sid = lax.axis_index("s")
    pltpu.sync_copy(c_hbm, cvec)
    pltpu.sync_copy(b_hbm, bvec)

    @pl.when(sid == 0)
    def _():
        pltpu.sync_copy(uxf_hbm, ux_sh)

    plsc.subcore_barrier()
    pltpu.sync_copy(ux_sh, uxf_v)
    cv = cvec[...]
    cs = [cv[m] for m in range(M)]
    b_lo = bvec[pl.ds(0, 16)]
    b_hi = bvec[pl.ds(16, 16)]

    def worker(base_w, per_w):
        n_pairs = per_w // C // 2
        pltpu.sync_copy(adjf_hbm.at[pl.ds(base_w * K, per_w * K)],
                        adj_all.at[pl.ds(0, per_w * K)])

        def build_idx(idx_ref, ci):
            for cc in range(C):
                a = adj_all[pl.ds((ci * C + cc) * K, K)]
                idx_ref[pl.ds(cc * K, K)] = jnp.maximum(a - 1, 0)

        def compute_chunk(wrows, ci):
            def node_body(cc, _):
                loc = ci * C + cc
                a = adj_all[pl.ds(loc * K, K)]
                valid = a > 0
                cnt = jnp.zeros((16,), jnp.float32) + jnp.sum(
                    jnp.where(valid, 1.0, 0.0))
                invc = jnp.where(cnt > 0.0, 1.0 / cnt, 0.0)
                idx0 = jnp.maximum(a - 1, 0)
                base4 = idx0 * M
                own = (base_w + loc) * M
                ps = []
                for m in range(M):
                    uxg = plsc.load_gather(uxf_v, [base4 + m])
                    uo = plsc.load_gather(
                        uxf_v, [jnp.full((16,), m, jnp.int32) + own])
                    ps.append((uo + cs[m]) - uxg)
                pmax = jnp.maximum(jnp.maximum(ps[0], ps[1]),
                                   jnp.maximum(ps[2], ps[3]))
                es = [jnp.exp(p - pmax) for p in ps]
                ssum = (es[0] + es[1]) + (es[2] + es[3])
                scale = invc / ssum
                wms = [jnp.where(valid, e * scale, 0.0) for e in es]
                acc_lo = b_lo
                acc_hi = b_hi
                for k in range(K):
                    j = cc * K + k
                    for m in range(M):
                        w = wms[m][k]
                        ab = wrows[j, pl.ds(32 * m, 32)]
                        lo, hi = plsc.unpack(
                            ab, format=plsc.PackFormat.INTERLEAVED)
                        acc_lo = acc_lo + w * lo
                        acc_hi = acc_hi + w * hi
                out_all[pl.ds(loc * COUT, 16)] = acc_lo
                out_all[pl.ds(loc * COUT + 16, 16)] = acc_hi
                return 0

            lax.fori_loop(0, C, node_body, 0)

        n_chunks = per_w // C
        n_outer = n_chunks // NBUF

        def fire(b, ci):
            build_idx(idx_list[b], ci)
            pltpu.async_copy(wx_hbm.at[idx_list[b]], wrows_list[b], sems[b])

        def ring_body(g, _):
            for b in range(NBUF):
                ci = g * NBUF + b
                pltpu.make_async_copy(
                    wx_hbm.at[idx_list[b]], wrows_list[b], sems[b]).wait()
                compute_chunk(wrows_list[b], ci)

                @pl.when(ci + NBUF < n_chunks)
                def _():
                    fire(b, ci + NBUF)

            return 0

        for b in range(NBUF):
            fire(b, b)
        lax.fori_loop(0, n_outer, ring_body, 0)
        pltpu.sync_copy(out_all.at[pl.ds(0, per_w * COUT)],
                        out_hbm.at[pl.ds(base_w * COUT, per_w * COUT)])

    @pl.when(cid == 0)
    def _():
        worker(sid * PER_W0, PER_W0)

    @pl.when(cid == 1)
    def _():
        worker(CORE1_BASE + sid * PER_W1, PER_W1)


_sc_kernel = functools.partial(
    pl.kernel,
    mesh=plsc.VectorSubcoreMesh(core_axis_name="c", subcore_axis_name="s"),
    compiler_params=pltpu.CompilerParams(needs_layout_passes=False,
                                         use_tc_tiling_on_sc=False),
    out_type=jax.ShapeDtypeStruct((N_PAD * COUT,), jnp.float32),
    scratch_types=[
        [pltpu.VMEM((C * K,), jnp.int32) for _ in range(NBUF)],
        pltpu.VMEM((PER_W_MAX * K,), jnp.int32),
        [pltpu.VMEM((C * K, CIN), jnp.bfloat16) for _ in range(NBUF)],
        pltpu.VMEM((N_PAD * M,), jnp.float32),
        pltpu.VMEM_SHARED((N_PAD * M,), jnp.float32),
        pltpu.VMEM((PER_W_MAX * COUT,), jnp.float32),
        pltpu.VMEM((16,), jnp.float32),
        pltpu.VMEM((COUT,), jnp.float32),
        [pltpu.SemaphoreType.DMA for _ in range(NBUF)],
    ],
)(_sc_body)


def kernel(x, adj, W, b, u, c):
    x2 = x[0]
    x2p = jnp.pad(x2, ((0, N_PAD - N), (0, 0)))
    Wr = W.reshape(M * COUT, CIN)
    perm = []
    for m in range(M):
        for i in range(16):
            perm.append(32 * m + i)
            perm.append(32 * m + 16 + i)
    perm = jnp.array(perm, dtype=jnp.int32)
    wx, z = _tc_matmul(x2p, Wr.T[:, perm], u.T)
    uxf = z.reshape(-1)
    adjf = jnp.pad(adj, ((0, N_PAD - N), (0, 0))).reshape(-1)
    c_pad = jnp.pad(c, (0, 16 - M))
    out = _sc_kernel(wx, uxf, adjf, c_pad, b)
    return out[:N * COUT].reshape(1, N, COUT)


# speedup vs baseline: 2.5762x; 1.3587x over previous
import functools

import jax
import jax.numpy as jnp
from jax import lax
from jax.experimental import pallas as pl
from jax.experimental.pallas import tpu as pltpu
from jax.experimental.pallas import tpu_sc as plsc

N = 10000
K = 16
CIN = 128
COUT = 32
M = 4

NW = 32
N_PAD = 10240
C = 8
NBUF = 2
PER_W0 = 496
PER_W1 = 144
PER_W1_STRIDE = 128
PER_W_MAX = PER_W0
CORE1_BASE = PER_W0 * 16


def _mm_body(x_ref, w_ref, u_ref, y_ref, z_ref):
    xv = x_ref[...]
    y_ref[...] = jnp.dot(xv, w_ref[...],
                         preferred_element_type=jnp.float32
                         ).astype(jnp.bfloat16)
    z_ref[...] = jnp.dot(xv, u_ref[...], preferred_element_type=jnp.float32)


def _tc_matmul(x2, wperm, ut):
    blk = 2000
    return pl.pallas_call(
        _mm_body,
        grid=(N // blk,),
        in_specs=[pl.BlockSpec((blk, CIN), lambda i: (i, 0)),
                  pl.BlockSpec((CIN, M * COUT), lambda i: (0, 0)),
                  pl.BlockSpec((CIN, M), lambda i: (0, 0))],
        out_specs=[pl.BlockSpec((blk, M * COUT), lambda i: (i, 0)),
                   pl.BlockSpec((blk, M), lambda i: (i, 0))],
        out_shape=[jax.ShapeDtypeStruct((N, M * COUT), jnp.bfloat16),
                   jax.ShapeDtypeStruct((N, M), jnp.float32)],
    )(x2, wperm, ut)


def _sc_body(wx_hbm, uxf_hbm, adjf_hbm, c_hbm, b_hbm, out_hbm,
             idx_list, adj_all, wrows_list, uxf_v, ux_sh, out_all,
             cvec, bvec, sems):
    cid = lax.axis_index("c")
    sid = lax.axis_index("s")
    pltpu.sync_copy(c_hbm, cvec)
    pltpu.sync_copy(b_hbm, bvec)

    @pl.when(sid == 0)
    def _():
        pltpu.sync_copy(uxf_hbm, ux_sh)

    plsc.subcore_barrier()
    pltpu.sync_copy(ux_sh, uxf_v)
    cv = cvec[...]
    cs = [cv[m] for m in range(M)]
    b_lo = bvec[pl.ds(0, 16)]
    b_hi = bvec[pl.ds(16, 16)]

    def worker(base_w, per_w):
        n_pairs = per_w // C // 2
        pltpu.sync_copy(adjf_hbm.at[pl.ds(base_w * K, per_w * K)],
                        adj_all.at[pl.ds(0, per_w * K)])

        def build_idx(idx_ref, ci):
            for cc in range(C):
                a = adj_all[pl.ds((ci * C + cc) * K, K)]
                idx_ref[pl.ds(cc * K, K)] = jnp.maximum(a - 1, 0)

        def compute_chunk(wrows, ci):
            def node_body(cc, _):
                loc = ci * C + cc
                a = adj_all[pl.ds(loc * K, K)]
                valid = a > 0
                cnt = jnp.zeros((16,), jnp.float32) + jnp.sum(
                    jnp.where(valid, 1.0, 0.0))
                invc = jnp.where(cnt > 0.0, 1.0 / cnt, 0.0)
                idx0 = jnp.maximum(a - 1, 0)
                base4 = idx0 * M
                own = (base_w + loc) * M
                ps = []
                for m in range(M):
                    uxg = plsc.load_gather(uxf_v, [base4 + m])
                    uo = plsc.load_gather(
                        uxf_v, [jnp.full((16,), m, jnp.int32) + own])
                    ps.append((uo + cs[m]) - uxg)
                pmax = jnp.maximum(jnp.maximum(ps[0], ps[1]),
                                   jnp.maximum(ps[2], ps[3]))
                es = [jnp.exp(p - pmax) for p in ps]
                ssum = (es[0] + es[1]) + (es[2] + es[3])
                scale = invc / ssum
                wms = [jnp.where(valid, e * scale, 0.0) for e in es]
                acc_lo = b_lo
                acc_hi = b_hi
                for k in range(K):
                    j = cc * K + k
                    for m in range(M):
                        w = wms[m][k]
                        ab = wrows[j, pl.ds(32 * m, 32)]
                        lo, hi = plsc.unpack(
                            ab, format=plsc.PackFormat.INTERLEAVED)
                        acc_lo = acc_lo + w * lo
                        acc_hi = acc_hi + w * hi
                out_all[pl.ds(loc * COUT, 16)] = acc_lo
                out_all[pl.ds(loc * COUT + 16, 16)] = acc_hi
                return 0

            lax.fori_loop(0, C, node_body, 0)

        n_chunks = per_w // C
        n_outer = n_chunks // NBUF

        def fire(b, ci):
            build_idx(idx_list[b], ci)
            pltpu.async_copy(wx_hbm.at[idx_list[b]], wrows_list[b], sems[b])

        def ring_body(g, _):
            for b in range(NBUF):
                ci = g * NBUF + b
                pltpu.make_async_copy(
                    wx_hbm.at[idx_list[b]], wrows_list[b], sems[b]).wait()
                compute_chunk(wrows_list[b], ci)

                @pl.when(ci + NBUF < n_chunks)
                def _():
                    fire(b, ci + NBUF)

            return 0

        for b in range(NBUF):
            fire(b, b)
        lax.fori_loop(0, n_outer, ring_body, 0)
        pltpu.sync_copy(out_all.at[pl.ds(0, per_w * COUT)],
                        out_hbm.at[pl.ds(base_w * COUT, per_w * COUT)])

    @pl.when(cid == 0)
    def _():
        worker(sid * PER_W0, PER_W0)

    @pl.when(cid == 1)
    def _():
        worker(CORE1_BASE + sid * PER_W1_STRIDE, PER_W1)


_sc_kernel = functools.partial(
    pl.kernel,
    mesh=plsc.VectorSubcoreMesh(core_axis_name="c", subcore_axis_name="s"),
    compiler_params=pltpu.CompilerParams(needs_layout_passes=False,
                                         use_tc_tiling_on_sc=False),
    out_type=jax.ShapeDtypeStruct((N * COUT,), jnp.float32),
    scratch_types=[
        [pltpu.VMEM((C * K,), jnp.int32) for _ in range(NBUF)],
        pltpu.VMEM((PER_W_MAX * K,), jnp.int32),
        [pltpu.VMEM((C * K, CIN), jnp.bfloat16) for _ in range(NBUF)],
        pltpu.VMEM((N * M,), jnp.float32),
        pltpu.VMEM_SHARED((N * M,), jnp.float32),
        pltpu.VMEM((PER_W_MAX * COUT,), jnp.float32),
        pltpu.VMEM((16,), jnp.float32),
        pltpu.VMEM((COUT,), jnp.float32),
        [pltpu.SemaphoreType.DMA for _ in range(NBUF)],
    ],
)(_sc_body)


def kernel(x, adj, W, b, u, c):
    x2 = x[0]
    Wr = W.reshape(M * COUT, CIN)
    perm = []
    for m in range(M):
        for i in range(16):
            perm.append(32 * m + i)
            perm.append(32 * m + 16 + i)
    perm = jnp.array(perm, dtype=jnp.int32)
    wx, z = _tc_matmul(x2, Wr.T[:, perm], u.T)
    uxf = z.reshape(-1)
    adjf = adj.reshape(-1)
    c_pad = jnp.pad(c, (0, 16 - M))
    out = _sc_kernel(wx, uxf, adjf, c_pad, b)
    return out.reshape(1, N, COUT)
